# baseline (speedup 1.0000x reference)
import jax
import jax.numpy as jnp
from jax import lax
from jax.experimental import pallas as pl
from jax.experimental.pallas import tpu as pltpu
from jax.experimental.pallas import tpu_sc as plsc

_B, _C, _H, _W, _K = 16, 8, 192, 192, 256
_HW = _H * _W
_NC, _NS, _L = 2, 16, 16
_NW = _NC * _NS
_PPW = (_B * _K) // _NW
_EPW = _PPW * _C
_CHUNK = 128
_NCHUNK = _EPW // _CHUNK
_NBUF = 4

_DNUMS = lax.GatherDimensionNumbers(
    offset_dims=(), collapsed_slice_dims=(0,), start_index_map=(0,))


def _permute(v, idx):
    return lax.gather(v, idx[:, None], _DNUMS, slice_sizes=(1,),
                      mode=lax.GatherScatterMode.PROMISE_IN_BOUNDS)


def _sc_body(out_flat, ind_flat, tgt_flat, msk_flat,
             red_out,
             ind_v, idx_v, pred_v, tgt_v, msk_v, res_v,
             gs0, gs1, gs2, gs3, csem):
    sems = (gs0, gs1, gs2, gs3)
    sid = lax.axis_index("s")
    cid = lax.axis_index("c")
    wid = sid * _NC + cid
    base = (wid // 2) * (_C * _HW)


    ct = pltpu.async_copy(tgt_flat.at[pl.ds(wid * _EPW, _EPW)], tgt_v, csem)
    cm = pltpu.async_copy(msk_flat.at[pl.ds(wid * _EPW, _EPW)], msk_v, csem)
    pltpu.sync_copy(ind_flat.at[pl.ds(wid * _PPW, _PPW)], ind_v)

    iota = lax.iota(jnp.int32, _L)
    c_off = (iota & 7) * _HW + base
    half = iota >> 3

    def build(g, carry):
        jv = ind_v[pl.ds(g * _L, _L)]
        for i in range(_L // 2):
            rep = _permute(jv, half + 2 * i)
            idx_v[pl.ds(g * _CHUNK + i * _L, _L)] = rep + c_off
        return carry

    lax.fori_loop(0, _NCHUNK, build, 0)

    def fire(g, slot):
        return pltpu.async_copy(
            out_flat.at[idx_v.at[pl.ds(g * _CHUNK, _CHUNK)]],
            pred_v.at[pl.ds(g * _CHUNK, _CHUNK)], sems[slot])

    for s in range(_NBUF):
        fire(s, s)
    ct.wait()
    cm.wait()

    def ring(gg, carry):
        acc_n, acc_d = carry
        for slot in range(_NBUF):
            g = gg * _NBUF + slot
            pltpu.make_async_copy(
                out_flat.at[pl.ds(0, _CHUNK)],
                pred_v.at[pl.ds(g * _CHUNK, _CHUNK)], sems[slot]).wait()

            @pl.when(gg < (_NCHUNK // _NBUF) - 1)
            def _():
                fire(g + _NBUF, slot)

            for i in range(_CHUNK // _L):
                off = g * _CHUNK + i * _L
                p = pred_v[pl.ds(off, _L)]
                tg = tgt_v[pl.ds(off, _L)]
                m = msk_v[pl.ds(off, _L)].astype(jnp.float32)
                acc_n = acc_n + jnp.abs(p * m - tg * m)
                acc_d = acc_d + m
        return acc_n, acc_d

    zero = jnp.zeros((_L,), jnp.float32)
    acc_n, acc_d = lax.fori_loop(0, _NCHUNK // _NBUF, ring, (zero, zero))

    rn = acc_n + lax.rev(acc_n, (0,))
    rd = acc_d + lax.rev(acc_d, (0,))
    res_v[...] = jnp.where(iota < 8, rn, rd)
    pltpu.sync_copy(res_v, red_out.at[wid])


def kernel(output, mask, ind, target, deps):
    del deps
    out_flat = output.reshape(-1)
    ind_flat = ind.reshape(-1)
    tgt_flat = target.reshape(-1)
    msk_flat = mask.reshape(-1)

    mesh = plsc.VectorSubcoreMesh(core_axis_name="c", subcore_axis_name="s")
    red = pl.kernel(
        _sc_body,
        mesh=mesh,
        out_type=jax.ShapeDtypeStruct((_NW, _L), jnp.float32),
        scratch_types=[
            pltpu.VMEM((_PPW,), jnp.int32),
            pltpu.VMEM((_EPW,), jnp.int32),
            pltpu.VMEM((_EPW,), jnp.float32),
            pltpu.VMEM((_EPW,), jnp.float32),
            pltpu.VMEM((_EPW,), jnp.int32),
            pltpu.VMEM((_L,), jnp.float32),
            pltpu.SemaphoreType.DMA,
            pltpu.SemaphoreType.DMA,
            pltpu.SemaphoreType.DMA,
            pltpu.SemaphoreType.DMA,
            pltpu.SemaphoreType.DMA,
        ],
    )(out_flat, ind_flat, tgt_flat, msk_flat)
    s = jnp.sum(red, axis=0)
    return jnp.sum(s[:8]) / (jnp.sum(s[8:]) + 0.0001)


# speedup vs baseline: 1.0057x; 1.0057x over previous
import jax
import jax.numpy as jnp
from jax import lax
from jax.experimental import pallas as pl
from jax.experimental.pallas import tpu as pltpu
from jax.experimental.pallas import tpu_sc as plsc

_B, _C, _H, _W, _K = 16, 8, 192, 192, 256
_HW = _H * _W
_NC, _NS, _L = 2, 16, 16
_NW = _NC * _NS
_PAIRS = _B * _K
_PPW = _PAIRS // _NW
_EPW = _PPW * _C
_CHUNK = 128
_NCHUNK = _EPW // _CHUNK


def _sc_body(out_flat, ind_flat, tgt_flat, msk_flat,
             red_out,
             ind_v, idx_v, pred_v, tgt_v, msk_v, res_v,
             gsem, csem):
    cid = lax.axis_index("c")
    sid = lax.axis_index("s")
    wid = sid * _NC + cid
    b = wid // (_K // _PPW)
    base = b * (_C * _HW)

    pltpu.sync_copy(ind_flat.at[pl.ds(wid * _PPW, _PPW)], ind_v)
    ct = pltpu.async_copy(tgt_flat.at[pl.ds(wid * _EPW, _EPW)], tgt_v, csem)
    cm = pltpu.async_copy(msk_flat.at[pl.ds(wid * _EPW, _EPW)], msk_v, csem)

    iota = lax.iota(jnp.int32, _L)
    c_off = (iota & 7) * _HW + base
    lo = iota < 8
    dnums = lax.GatherDimensionNumbers(
        offset_dims=(), collapsed_slice_dims=(0,), start_index_map=(0,))
    half = iota >> 3

    def build(u, carry):
        jv = ind_v[pl.ds(u * _L, _L)]
        for i in range(_L // 2):
            rep = lax.gather(jv, (half + 2 * i)[:, None], dnums,
                             slice_sizes=(1,),
                             mode=lax.GatherScatterMode.PROMISE_IN_BOUNDS)
            idx_v[pl.ds(u * (_L * 8) + i * _L, _L)] = rep + c_off
        return carry

    lax.fori_loop(0, _PPW // _L, build, 0)

    copies = [
        pltpu.async_copy(out_flat.at[idx_v.at[pl.ds(g * _CHUNK, _CHUNK)]],
                         pred_v.at[pl.ds(g * _CHUNK, _CHUNK)], gsem)
        for g in range(_NCHUNK)
    ]
    for cp in copies:
        cp.wait()
    ct.wait()
    cm.wait()

    def body(t, carry):
        acc_n, acc_d = carry
        p = pred_v[pl.ds(t * _L, _L)]
        tg = tgt_v[pl.ds(t * _L, _L)]
        m = msk_v[pl.ds(t * _L, _L)].astype(jnp.float32)
        return acc_n + jnp.abs(p * m - tg * m), acc_d + m

    zero = jnp.zeros((_L,), jnp.float32)
    acc_n, acc_d = lax.fori_loop(0, _EPW // _L, body, (zero, zero))

    rn = acc_n + lax.rev(acc_n, (0,))
    rd = acc_d + lax.rev(acc_d, (0,))
    res_v[...] = jnp.where(lo, rn, rd)
    pltpu.sync_copy(res_v, red_out.at[wid])


def kernel(output, mask, ind, target, deps):
    del deps
    out_flat = output.reshape(-1)
    ind_flat = ind.reshape(-1)
    tgt_flat = target.reshape(-1)
    msk_flat = mask.reshape(-1)

    mesh = plsc.VectorSubcoreMesh(core_axis_name="c", subcore_axis_name="s")
    red = pl.kernel(
        _sc_body,
        mesh=mesh,
        out_type=jax.ShapeDtypeStruct((_NW, _L), jnp.float32),
        scratch_types=[
            pltpu.VMEM((_PPW,), jnp.int32),
            pltpu.VMEM((_EPW,), jnp.int32),
            pltpu.VMEM((_EPW,), jnp.float32),
            pltpu.VMEM((_EPW,), jnp.float32),
            pltpu.VMEM((_EPW,), jnp.int32),
            pltpu.VMEM((_L,), jnp.float32),
            pltpu.SemaphoreType.DMA,
            pltpu.SemaphoreType.DMA,
        ],
    )(out_flat, ind_flat, tgt_flat, msk_flat)
    s = jnp.sum(red, axis=0)
    return jnp.sum(s[:8]) / (jnp.sum(s[8:]) + 0.0001)
